# Initial kernel scaffold; baseline (speedup 1.0000x reference)
import jax
import jax.numpy as jnp
from jax.experimental import pallas as pl


def kernel(bit_sequence, matrix):
    raise NotImplementedError("write your pallas kernel here")


# speedup vs baseline: 3.0763x; 3.0763x over previous
import functools

import jax
import jax.numpy as jnp
from jax import lax
from jax.experimental import pallas as pl
from jax.experimental.pallas import tpu as pltpu
from jax.experimental.pallas import tpu_sc as plsc

_L = 16


def _encoder_body(B, W, M, NC, NS, C, bits_hbm, tbl_hbm, out_hbm,
                  tbl_v, bits_v, out_v):
    NW = NC * NS
    RW = B // NW
    NCH = RW // C
    wid = lax.axis_index("s") * NC + lax.axis_index("c")

    pltpu.sync_copy(tbl_hbm, tbl_v)
    nvec = (2 * M) // _L

    def _ssq_body(i, acc):
        v = tbl_v[pl.ds(i * _L, _L)]
        return acc + v * v

    ssq = lax.fori_loop(0, nvec, _ssq_body, jnp.zeros((_L,), jnp.float32))
    mean = jnp.sum(ssq) * jnp.float32(1.0 / M)
    mv = lax.broadcast_in_dim(mean, (_L,), ())
    ii = plsc.bitcast(mv, jnp.int32)
    ii = jnp.int32(0x5F3759DF) - (ii >> 1)
    y = plsc.bitcast(ii, jnp.float32)
    half = mv * jnp.float32(0.5)
    for _ in range(4):
        y = y * (jnp.float32(1.5) - half * y * y)
    inv_nf = y

    def _scale_body(i, _):
        tbl_v[pl.ds(i * _L, _L)] = tbl_v[pl.ds(i * _L, _L)] * inv_nf
        return 0

    lax.fori_loop(0, nvec, _scale_body, 0)

    iota = lax.iota(jnp.int32, _L)
    iota_w = iota * W
    iota_2 = iota * 2
    base_row = wid * RW

    def _chunk(g, _):
        row0 = base_row + g * C
        pltpu.sync_copy(bits_hbm.at[pl.ds(row0 * W, C * W)], bits_v)

        def _group(j, _c):
            bidx = iota_w + j * (_L * W)
            acc = plsc.load_gather(bits_v, [bidx])
            for k in range(1, W):
                bv = plsc.load_gather(bits_v, [bidx + k])
                acc = acc + acc + bv
            idx = acc.astype(jnp.int32)
            re = plsc.load_gather(tbl_v, [idx * 2])
            im = plsc.load_gather(tbl_v, [idx * 2 + 1])
            oidx = iota_2 + j * (_L * 2)
            plsc.store_scatter(out_v, [oidx], re)
            plsc.store_scatter(out_v, [oidx + 1], im)
            return 0

        lax.fori_loop(0, C // _L, _group, 0)
        pltpu.sync_copy(out_v, out_hbm.at[pl.ds(row0 * 2, C * 2)])
        return 0

    lax.fori_loop(0, NCH, _chunk, 0)


@functools.partial(jax.jit, static_argnums=(2, 3))
def _encode(bits_flat, tbl_flat, B, W):
    M = tbl_flat.shape[0] // 2
    info = plsc.get_sparse_core_info()
    NC, NS = info.num_cores, info.num_subcores
    C = 2048
    mesh = plsc.VectorSubcoreMesh(core_axis_name="c", subcore_axis_name="s")
    k = pl.kernel(
        functools.partial(_encoder_body, B, W, M, NC, NS, C),
        mesh=mesh,
        compiler_params=pltpu.CompilerParams(needs_layout_passes=False),
        out_type=jax.ShapeDtypeStruct((B * 2,), jnp.float32),
        scratch_types=[
            pltpu.VMEM((2 * M,), jnp.float32),
            pltpu.VMEM((C * W,), jnp.float32),
            pltpu.VMEM((C * 2,), jnp.float32),
        ],
    )
    return k(bits_flat, tbl_flat)


def kernel(bit_sequence, matrix):
    B, W = bit_sequence.shape
    out = _encode(bit_sequence.reshape(B * W), matrix.reshape(-1), B, W)
    return out.reshape(B, 2)
